# baseline (speedup 1.0000x reference)
import functools

import jax
import jax.numpy as jnp
from jax import lax
from jax.experimental import pallas as pl
from jax.experimental.pallas import tpu as pltpu
from jax.experimental.pallas import tpu_sc as plsc

EMBED_DIM = 32
CHUNK = 128


@functools.lru_cache(maxsize=None)
def _build(B: int):
    info = plsc.get_sparse_core_info()
    NC, NS = info.num_cores, info.num_subcores
    NW = NC * NS
    b_per_w = B // NW
    n_chunks = b_per_w // CHUNK

    mesh = plsc.VectorSubcoreMesh(core_axis_name="c", subcore_axis_name="s")

    @functools.partial(
        pl.kernel,
        mesh=mesh,
        compiler_params=pltpu.CompilerParams(needs_layout_passes=False),
        out_type=jax.ShapeDtypeStruct((B,), jnp.float32),
        scratch_types=[
            pltpu.VMEM((b_per_w,), jnp.int32),
            pltpu.VMEM((b_per_w,), jnp.int32),
            pltpu.VMEM((b_per_w,), jnp.int32),
            pltpu.VMEM((b_per_w,), jnp.int32),
            pltpu.VMEM((CHUNK, 128), jnp.float32),
            pltpu.VMEM((CHUNK, 128), jnp.float32),
            pltpu.VMEM((EMBED_DIM,), jnp.float32),
            pltpu.VMEM((16,), jnp.float32),
            pltpu.VMEM((b_per_w,), jnp.float32),
            pltpu.VMEM((17 * 16,), jnp.float32),
            pltpu.SemaphoreType.DMA,
        ],
    )
    def kfn(uid_hbm, bid_hbm, utab_hbm, btab_hbm, w_hbm, b_hbm, out_hbm,
            uid_v, bid_v, uq_v, bq_v, ubuf_v, bbuf_v, w_v, b_v, out_v, tr_v,
            gsem):
        wid = lax.axis_index("s") * NC + lax.axis_index("c")
        base = wid * b_per_w

        pltpu.sync_copy(uid_hbm.at[pl.ds(base, b_per_w)], uid_v)
        pltpu.sync_copy(bid_hbm.at[pl.ds(base, b_per_w)], bid_v)
        pltpu.sync_copy(w_hbm, w_v)
        pltpu.sync_copy(b_hbm, b_v)

        def split_body(g, carry):
            sl = pl.ds(g * 16, 16)
            for src, qdst in ((uid_v, uq_v), (bid_v, bq_v)):
                v = src[sl]
                qdst[sl] = lax.shift_right_logical(v, 2)
                src[sl] = lax.shift_left(v & 3, 5)
            return carry

        lax.fori_loop(0, b_per_w // 16, split_body, 0)

        w0 = w_v[pl.ds(0, 16)]
        w1 = w_v[pl.ds(16, 16)]
        fcb = b_v[pl.ds(0, 16)][0]
        col_base = lax.iota(jnp.int32, 16) * 17

        for c in range(n_chunks):
            sl = pl.ds(c * CHUNK, CHUNK)
            cu = pltpu.async_copy(utab_hbm.at[uq_v.at[sl]], ubuf_v, gsem)
            cb = pltpu.async_copy(btab_hbm.at[bq_v.at[sl]], bbuf_v, gsem)
            cu.wait()
            cb.wait()

            def group_body(g, carry, c=c):
                r0 = g * 16
                ov = uid_v[pl.ds(c * CHUNK + r0, 16)]
                bv = bid_v[pl.ds(c * CHUNK + r0, 16)]
                for r in range(16):
                    uo = ov[r]
                    bo = bv[r]
                    u0 = ubuf_v[r0 + r, pl.ds(uo, 16)]
                    u1 = ubuf_v[r0 + r, pl.ds(uo + 16, 16)]
                    bb0 = bbuf_v[r0 + r, pl.ds(bo, 16)]
                    bb1 = bbuf_v[r0 + r, pl.ds(bo + 16, 16)]
                    p = u0 * bb0 * w0 + u1 * bb1 * w1
                    plsc.store_scatter(tr_v, [col_base + r], p)
                acc = jnp.full((16,), fcb, dtype=jnp.float32)
                for d in range(16):
                    acc = acc + tr_v[pl.ds(d * 17, 16)]
                out_v[pl.ds(c * CHUNK + r0, 16)] = acc
                return carry

            lax.fori_loop(0, CHUNK // 16, group_body, 0)

        pltpu.sync_copy(out_v, out_hbm.at[pl.ds(base, b_per_w)])

    return kfn


def kernel(user_id, book_id, user_table, book_table, fc_w, fc_b):
    B = user_id.shape[0]
    nu = user_table.shape[0]
    nb = book_table.shape[0]
    pu = (-nu) % 4
    pb = (-nb) % 4
    u128 = jnp.pad(user_table, ((0, pu), (0, 0))).reshape((nu + pu) // 4, 128)
    b128 = jnp.pad(book_table, ((0, pb), (0, 0))).reshape((nb + pb) // 4, 128)
    w = fc_w.reshape(EMBED_DIM)
    b = jnp.pad(fc_b, (0, 15))
    return _build(B)(user_id.astype(jnp.int32), book_id.astype(jnp.int32),
                     u128, b128, w, b)
